# baseline (speedup 1.0000x reference)
import functools
import math

import jax
import jax.numpy as jnp
from jax import lax
from jax.experimental import pallas as pl
from jax.experimental.pallas import tpu as pltpu
from jax.experimental.pallas import tpu_sc as plsc

BATCH = 16384
EMB = 64

_NC = 2
_NS = 16
_NW = _NC * _NS


def _gather_body(uids_hbm, iids_hbm, ut_hbm, it_hbm, ue_hbm, ie_hbm,
                 uidx_v, urows_v, iidx_v, irows_v, sem_u, sem_i, bpw):
    wid = lax.axis_index("s") * _NC + lax.axis_index("c")
    base = wid * bpw
    pltpu.sync_copy(uids_hbm.at[pl.ds(base, bpw)], uidx_v)
    pltpu.sync_copy(iids_hbm.at[pl.ds(base, bpw)], iidx_v)
    cu = pltpu.async_copy(ut_hbm.at[uidx_v], urows_v, sem_u)
    ci = pltpu.async_copy(it_hbm.at[iidx_v], irows_v, sem_i)
    cu.wait()
    pltpu.sync_copy(urows_v, ue_hbm.at[pl.ds(base, bpw)])
    ci.wait()
    pltpu.sync_copy(irows_v, ie_hbm.at[pl.ds(base, bpw)])


def _sc_gather(user_ids, item_ids, user_table, item_table):
    bpw = BATCH // _NW
    mesh = plsc.VectorSubcoreMesh(core_axis_name="c", subcore_axis_name="s")
    out_type = [
        jax.ShapeDtypeStruct((BATCH, EMB), jnp.float32),
        jax.ShapeDtypeStruct((BATCH, EMB), jnp.float32),
    ]
    scratch = [
        pltpu.VMEM((bpw,), jnp.int32),
        pltpu.VMEM((bpw, EMB), jnp.float32),
        pltpu.VMEM((bpw,), jnp.int32),
        pltpu.VMEM((bpw, EMB), jnp.float32),
        pltpu.SemaphoreType.DMA,
        pltpu.SemaphoreType.DMA,
    ]
    k = pl.kernel(
        functools.partial(_gather_body, bpw=bpw),
        out_type=out_type,
        mesh=mesh,
        scratch_types=scratch,
        compiler_params=pltpu.CompilerParams(use_tc_tiling_on_sc=False),
    )
    return k(user_ids, item_ids, user_table, item_table)


def _layernorm(x, g, b, eps=1e-5):
    mu = jnp.mean(x, axis=-1, keepdims=True)
    var = jnp.mean((x - mu) ** 2, axis=-1, keepdims=True)
    return (x - mu) / jnp.sqrt(var + eps) * g + b


def _gelu(x):
    return 0.5 * x * (1.0 + lax.erf(x * (1.0 / math.sqrt(2.0))))


def _mlp_body(ue, ie, W0u, W0i, b0, g0, beta0, W1, b1, g1, beta1,
              W2, b2, g2, beta2, W_out, b_out, out):
    dot = functools.partial(jnp.dot, preferred_element_type=jnp.float32,
                            precision=lax.Precision.HIGHEST)
    x = dot(ue[...], W0u[...]) + dot(ie[...], W0i[...]) + b0[...]
    x = _gelu(_layernorm(x, g0[...], beta0[...]))
    x = dot(x, W1[...]) + b1[...]
    x = _gelu(_layernorm(x, g1[...], beta1[...]))
    x = dot(x, W2[...]) + b2[...]
    x = _gelu(_layernorm(x, g2[...], beta2[...]))
    out[...] = dot(x, W_out[...]) + b_out[...]


def _tc_mlp(ue, ie, W0, b0, g0, beta0, W1, b1, g1, beta1,
            W2, b2, g2, beta2, W_out, b_out):
    blk = 2048
    grid = (BATCH // blk,)
    W0u = W0[:EMB]
    W0i = W0[EMB:]

    def row_spec(n):
        return pl.BlockSpec((blk, n), lambda i: (i, 0))

    def full_spec(a):
        return pl.BlockSpec(a.shape, lambda i: (0,) * a.ndim)

    b0r, g0r, beta0r = (a.reshape(1, -1) for a in (b0, g0, beta0))
    b1r, g1r, beta1r = (a.reshape(1, -1) for a in (b1, g1, beta1))
    b2r, g2r, beta2r = (a.reshape(1, -1) for a in (b2, g2, beta2))
    b_outr = b_out.reshape(1, -1)

    args = (ue, ie, W0u, W0i, b0r, g0r, beta0r, W1, b1r, g1r, beta1r,
            W2, b2r, g2r, beta2r, W_out, b_outr)
    in_specs = [row_spec(EMB), row_spec(EMB)] + [full_spec(a) for a in args[2:]]
    return pl.pallas_call(
        _mlp_body,
        grid=grid,
        in_specs=in_specs,
        out_specs=pl.BlockSpec((blk, 1), lambda i: (i, 0)),
        out_shape=jax.ShapeDtypeStruct((BATCH, 1), jnp.float32),
    )(*args)


def kernel(user_ids, item_ids, user_table, item_table,
           W0, b0, g0, beta0, W1, b1, g1, beta1, W2, b2, g2, beta2,
           W_out, b_out):
    ue, ie = _sc_gather(user_ids.astype(jnp.int32), item_ids.astype(jnp.int32),
                        user_table, item_table)
    return _tc_mlp(ue, ie, W0, b0, g0, beta0, W1, b1, g1, beta1,
                   W2, b2, g2, beta2, W_out, b_out)
